# Initial kernel scaffold; baseline (speedup 1.0000x reference)
import jax
import jax.numpy as jnp
from jax.experimental import pallas as pl


def kernel(x, edge_index, edge_attr, W1, b1, W2, b2, eps, W3, b3, W4, b4):
    raise NotImplementedError("write your pallas kernel here")


# speedup vs baseline: 1.8268x; 1.8268x over previous
import functools

import jax
import jax.numpy as jnp
from jax import lax
from jax.experimental import pallas as pl
from jax.experimental.pallas import tpu as pltpu
from jax.experimental.pallas import tpu_sc as plsc

_HI = lax.Precision.HIGHEST


def _lrelu(v):
    return jnp.where(v >= 0, v, 0.01 * v)


def _gates_body(ea_ref, w1t_ref, b1_ref, w2b_ref, b2_ref, out_ref):
    ea = ea_ref[...]
    h = _lrelu(lax.dot(ea, w1t_ref[...], precision=_HI,
                       preferred_element_type=jnp.float32) + b1_ref[...])
    gp = lax.dot(h, w2b_ref[...], precision=_HI,
                 preferred_element_type=jnp.float32) + b2_ref[...]
    g = jnp.where(gp >= 0, gp, jnp.exp(jnp.minimum(gp, 0.0)) - 1.0)
    parts = [jnp.broadcast_to(g[:, c:c + 1], (g.shape[0], 16)) for c in range(4)]
    out_ref[...] = jnp.concatenate(parts, axis=1)


def _gates(ea, w1t, b1s, w2b, b2r):
    E = ea.shape[0]
    BE = 8000
    grid = E // BE
    return pl.pallas_call(
        _gates_body,
        grid=(grid,),
        in_specs=[
            pl.BlockSpec((BE, 4), lambda i: (i, 0)),
            pl.BlockSpec((4, 32), lambda i: (0, 0)),
            pl.BlockSpec((1, 32), lambda i: (0, 0)),
            pl.BlockSpec((32, 4), lambda i: (0, 0)),
            pl.BlockSpec((1, 4), lambda i: (0, 0)),
        ],
        out_specs=pl.BlockSpec((BE, 64), lambda i: (i, 0)),
        out_shape=jax.ShapeDtypeStruct((E, 64), jnp.float32),
    )(ea, w1t, b1s, w2b, b2r)


def _y_body(x_ref, wcat_ref, y_ref):
    y_ref[...] = lax.dot(x_ref[...], wcat_ref[...], precision=_HI,
                         preferred_element_type=jnp.float32)


def _y_proj(x, wcat):
    N, D = x.shape
    BN = 1000
    grid = N // BN
    return pl.pallas_call(
        _y_body,
        grid=(grid,),
        in_specs=[
            pl.BlockSpec((BN, D), lambda i: (i, 0)),
            pl.BlockSpec((D, 4 * 128), lambda i: (0, 0)),
        ],
        out_specs=pl.BlockSpec((BN, 4 * 128), lambda i: (i, 0)),
        out_shape=jax.ShapeDtypeStruct((N, 4 * 128), jnp.float32),
    )(x, wcat)


def _sc_spmm(y, src, dst, wbc, n, d):
    E = src.shape[0]
    info = plsc.get_sparse_core_info()
    NC, NS = info.num_cores, info.num_subcores
    NW = NC * NS
    EPW = E // NW
    K = 40
    STEPS = EPW // K
    zeros = jnp.zeros((n, d), jnp.float32)

    mesh = plsc.VectorSubcoreMesh(core_axis_name="c", subcore_axis_name="s")

    @functools.partial(
        pl.kernel,
        out_type=jax.ShapeDtypeStruct((NC * n, d), jnp.float32),
        mesh=mesh,
        scratch_types=[
            pltpu.VMEM((K,), jnp.int32),
            pltpu.VMEM((K,), jnp.int32),
            pltpu.VMEM((K, 64), jnp.float32),
            pltpu.VMEM((K, 4 * d), jnp.float32),
            pltpu.VMEM((K, d), jnp.float32),
            pltpu.VMEM_SHARED((n, d), jnp.float32),
            pltpu.SemaphoreType.DMA,
        ],
    )
    def spmm(y_hbm, src_hbm, dst_hbm, wbc_hbm, z_hbm, out_hbm,
             srcv, dstv, wv, rows, zb, accum, sem):
        c = lax.axis_index("c")
        s = lax.axis_index("s")
        base = (s * NC + c) * EPW

        @pl.when(s < NS - 1)
        def _():
            pltpu.sync_copy(z_hbm.at[pl.ds(s * 640, 640)],
                            accum.at[pl.ds(s * 640, 640)])

        @pl.when(s == NS - 1)
        def _():
            pltpu.sync_copy(z_hbm.at[pl.ds((NS - 1) * 640, n - (NS - 1) * 640)],
                            accum.at[pl.ds((NS - 1) * 640, n - (NS - 1) * 640)])

        plsc.subcore_barrier()

        def step(i, carry):
            off = base + i * K
            pltpu.sync_copy(src_hbm.at[pl.ds(off, K)], srcv)
            pltpu.sync_copy(dst_hbm.at[pl.ds(off, K)], dstv)
            pltpu.sync_copy(wbc_hbm.at[pl.ds(off, K)], wv)
            pltpu.async_copy(y_hbm.at[srcv], rows, sem).wait()

            def edge(e, cc):
                w0 = wv[e, pl.ds(0, 16)]
                w1 = wv[e, pl.ds(16, 16)]
                w2 = wv[e, pl.ds(32, 16)]
                w3 = wv[e, pl.ds(48, 16)]
                for j in range(d // 16):
                    z = (rows[e, pl.ds(j * 16, 16)] * w0
                         + rows[e, pl.ds(d + j * 16, 16)] * w1
                         + rows[e, pl.ds(2 * d + j * 16, 16)] * w2
                         + rows[e, pl.ds(3 * d + j * 16, 16)] * w3)
                    zb[e, pl.ds(j * 16, 16)] = z
                return cc

            lax.fori_loop(0, K, edge, 0)
            pltpu.sync_copy(zb, accum.at[dstv], add=True)
            return carry

        lax.fori_loop(0, STEPS, step, 0)
        plsc.subcore_barrier()

        @pl.when(s < NS - 1)
        def _():
            pltpu.sync_copy(accum.at[pl.ds(s * 640, 640)],
                            out_hbm.at[pl.ds(c * n + s * 640, 640)])

        @pl.when(s == NS - 1)
        def _():
            pltpu.sync_copy(
                accum.at[pl.ds((NS - 1) * 640, n - (NS - 1) * 640)],
                out_hbm.at[pl.ds(c * n + (NS - 1) * 640, n - (NS - 1) * 640)])

    return spmm(y, src, dst, wbc, zeros)


def _final_body(h0_ref, h1_ref, y_ref, epsf_ref, b3_ref, w4t_ref, b4_ref, out_ref):
    yb = y_ref[...] * epsf_ref[...]
    base = (yb[:, 0:128] + yb[:, 128:256]
            + yb[:, 256:384] + yb[:, 384:512])
    pre = h0_ref[...] + h1_ref[...] + base + b3_ref[...]
    h = _lrelu(pre)
    out_ref[...] = lax.dot(h, w4t_ref[...], precision=_HI,
                           preferred_element_type=jnp.float32) + b4_ref[...]


def _final(h0, h1, y, epsf, b3r, w4t, b4r):
    N, D = h0.shape
    BN = 1000
    grid = N // BN
    return pl.pallas_call(
        _final_body,
        grid=(grid,),
        in_specs=[
            pl.BlockSpec((BN, D), lambda i: (i, 0)),
            pl.BlockSpec((BN, D), lambda i: (i, 0)),
            pl.BlockSpec((BN, 4 * D), lambda i: (i, 0)),
            pl.BlockSpec((1, 4 * D), lambda i: (0, 0)),
            pl.BlockSpec((1, D), lambda i: (0, 0)),
            pl.BlockSpec((D, D), lambda i: (0, 0)),
            pl.BlockSpec((1, D), lambda i: (0, 0)),
        ],
        out_specs=pl.BlockSpec((BN, D), lambda i: (i, 0)),
        out_shape=jax.ShapeDtypeStruct((N, D), jnp.float32),
    )(h0, h1, y, epsf, b3r, w4t, b4r)


def kernel(x, edge_index, edge_attr, W1, b1, W2, b2, eps, W3, b3, W4, b4):
    N, D = x.shape
    C = W1.shape[0]
    OUT = W3.shape[0]

    w1t = W1.reshape(C * 8, 4).T
    b1s = b1.reshape(1, C * 8)
    w2b = (jnp.eye(C, dtype=jnp.float32)[:, None, :]
           * W2.reshape(C, 8)[:, :, None]).reshape(C * 8, C)
    b2r = b2.reshape(1, C)
    wcat = W3.reshape(OUT, C, D).transpose(2, 1, 0).reshape(D, C * OUT)
    epsf = jnp.repeat(1.0 + eps, OUT).reshape(1, C * OUT)
    b3r = b3.reshape(1, OUT)
    w4t = W4.T
    b4r = b4.reshape(1, OUT)

    wbc = _gates(edge_attr, w1t, b1s, w2b, b2r)
    y = _y_proj(x, wcat)
    src = edge_index[0]
    dst = edge_index[1]
    partials = _sc_spmm(y, src, dst, wbc, N, OUT)
    h0 = partials[:N]
    h1 = partials[N:]
    return _final(h0, h1, y, epsf, b3r, w4t, b4r)


# speedup vs baseline: 2.7653x; 1.5138x over previous
import functools

import jax
import jax.numpy as jnp
from jax import lax
from jax.experimental import pallas as pl
from jax.experimental.pallas import tpu as pltpu
from jax.experimental.pallas import tpu_sc as plsc

_HI = lax.Precision.HIGHEST


def _lrelu(v):
    return jnp.where(v >= 0, v, 0.01 * v)


def _gates_body(ea_ref, w1t_ref, b1_ref, w2b_ref, b2_ref, out_ref):
    ea = ea_ref[...]
    h = _lrelu(lax.dot(ea, w1t_ref[...], precision=_HI,
                       preferred_element_type=jnp.float32) + b1_ref[...])
    gp = lax.dot(h, w2b_ref[...], precision=_HI,
                 preferred_element_type=jnp.float32) + b2_ref[...]
    g = jnp.where(gp >= 0, gp, jnp.exp(jnp.minimum(gp, 0.0)) - 1.0)
    parts = [jnp.broadcast_to(g[:, c:c + 1], (g.shape[0], 16)) for c in range(4)]
    out_ref[...] = jnp.concatenate(parts, axis=1)


def _gates(ea, w1t, b1s, w2b, b2r):
    E = ea.shape[0]
    BE = 8000
    grid = E // BE
    return pl.pallas_call(
        _gates_body,
        grid=(grid,),
        in_specs=[
            pl.BlockSpec((BE, 4), lambda i: (i, 0)),
            pl.BlockSpec((4, 32), lambda i: (0, 0)),
            pl.BlockSpec((1, 32), lambda i: (0, 0)),
            pl.BlockSpec((32, 4), lambda i: (0, 0)),
            pl.BlockSpec((1, 4), lambda i: (0, 0)),
        ],
        out_specs=pl.BlockSpec((BE, 64), lambda i: (i, 0)),
        out_shape=jax.ShapeDtypeStruct((E, 64), jnp.float32),
    )(ea, w1t, b1s, w2b, b2r)


def _y_body(x_ref, wcat_ref, y_ref):
    y_ref[...] = lax.dot(x_ref[...], wcat_ref[...], precision=_HI,
                         preferred_element_type=jnp.float32)


def _y_proj(x, wcat):
    N, D = x.shape
    BN = 1000
    grid = N // BN
    return pl.pallas_call(
        _y_body,
        grid=(grid,),
        in_specs=[
            pl.BlockSpec((BN, D), lambda i: (i, 0)),
            pl.BlockSpec((D, 4 * 128), lambda i: (0, 0)),
        ],
        out_specs=pl.BlockSpec((BN, 4 * 128), lambda i: (i, 0)),
        out_shape=jax.ShapeDtypeStruct((N, 4 * 128), jnp.float32),
    )(x, wcat)


_K = 32
_PAD = 10048


def _sc_spmm(y, rec, wbc, n, d):
    info = plsc.get_sparse_core_info()
    NC, NS = info.num_cores, info.num_subcores
    K = _K
    STEPS = _PAD // K
    PAIRS = STEPS // 2
    zeros = jnp.zeros((n, d), jnp.float32)

    mesh = plsc.VectorSubcoreMesh(core_axis_name="c", subcore_axis_name="s")

    @functools.partial(
        pl.kernel,
        out_type=jax.ShapeDtypeStruct((NC * n, d), jnp.float32),
        mesh=mesh,
        scratch_types=[
            pltpu.VMEM((2, 2, K), jnp.int32),
            pltpu.VMEM((2, K, 64), jnp.float32),
            pltpu.VMEM((2, K, 4 * d), jnp.float32),
            pltpu.VMEM((K, d), jnp.float32),
            pltpu.VMEM_SHARED((n, d), jnp.float32),
            pltpu.SemaphoreType.DMA,
            pltpu.SemaphoreType.DMA,
            pltpu.SemaphoreType.DMA,
            pltpu.SemaphoreType.DMA,
        ],
    )
    def spmm(y_hbm, rec_hbm, wbc_hbm, z_hbm, out_hbm,
             recv, wv, rows, zb, accum, si0, si1, sg0, sg1):
        c = lax.axis_index("c")
        s = lax.axis_index("s")
        wid = s * NC + c
        sbase = wid * STEPS
        si = (si0, si1)
        sg = (sg0, sg1)

        @pl.when(s < NS - 1)
        def _():
            pltpu.sync_copy(z_hbm.at[pl.ds(s * 640, 640)],
                            accum.at[pl.ds(s * 640, 640)])

        @pl.when(s == NS - 1)
        def _():
            pltpu.sync_copy(z_hbm.at[pl.ds((NS - 1) * 640, n - (NS - 1) * 640)],
                            accum.at[pl.ds((NS - 1) * 640, n - (NS - 1) * 640)])

        plsc.subcore_barrier()

        def idx_fetch(step, b):
            gt = sbase + step
            pltpu.async_copy(rec_hbm.at[gt], recv.at[b], si[b])
            pltpu.async_copy(wbc_hbm.at[pl.ds(gt * K, K)], wv.at[b], si[b])

        def wait_idx(b):
            pltpu.make_async_copy(rec_hbm.at[0], recv.at[b], si[b]).wait()
            pltpu.make_async_copy(wbc_hbm.at[pl.ds(0, K)], wv.at[b],
                                  si[b]).wait()

        def gather(b):
            pltpu.async_copy(y_hbm.at[recv.at[b, 0]], rows.at[b], sg[b])

        def wait_gather(b):
            pltpu.make_async_copy(y_hbm.at[pl.ds(0, K)], rows.at[b],
                                  sg[b]).wait()

        def compute(b):
            @plsc.parallel_loop(0, K, step=1, unroll=4)
            def _(e):
                w0 = wv[b, e, pl.ds(0, 16)]
                w1 = wv[b, e, pl.ds(16, 16)]
                w2 = wv[b, e, pl.ds(32, 16)]
                w3 = wv[b, e, pl.ds(48, 16)]
                for j in range(d // 16):
                    z = (rows[b, e, pl.ds(j * 16, 16)] * w0
                         + rows[b, e, pl.ds(d + j * 16, 16)] * w1
                         + rows[b, e, pl.ds(2 * d + j * 16, 16)] * w2
                         + rows[b, e, pl.ds(3 * d + j * 16, 16)] * w3)
                    zb[e, pl.ds(j * 16, 16)] = z

        def scatter(b):
            pltpu.sync_copy(zb, accum.at[recv.at[b, 1]], add=True)

        def half(t, b, last_pair):
            if b == 0:
                wait_idx(1)
                gather(1)
            else:
                @pl.when(jnp.logical_not(last_pair))
                def _():
                    wait_idx(0)
                    gather(0)
            wait_gather(b)
            compute(b)
            scatter(b)

            @pl.when(jnp.logical_not(last_pair))
            def _():
                idx_fetch(t + 2, b)

        idx_fetch(0, 0)
        idx_fetch(1, 1)
        wait_idx(0)
        gather(0)

        def pair(i2, carry):
            t = 2 * i2
            last = i2 >= PAIRS - 1
            half(t, 0, last)
            half(t + 1, 1, last)
            return carry

        lax.fori_loop(0, PAIRS, pair, 0)
        plsc.subcore_barrier()

        @pl.when(s < NS - 1)
        def _():
            pltpu.sync_copy(accum.at[pl.ds(s * 640, 640)],
                            out_hbm.at[pl.ds(c * n + s * 640, 640)])

        @pl.when(s == NS - 1)
        def _():
            pltpu.sync_copy(
                accum.at[pl.ds((NS - 1) * 640, n - (NS - 1) * 640)],
                out_hbm.at[pl.ds(c * n + (NS - 1) * 640, n - (NS - 1) * 640)])

    return spmm(y, rec, wbc, zeros)


def _final_body(h0_ref, h1_ref, y_ref, epsf_ref, b3_ref, w4t_ref, b4_ref, out_ref):
    yb = y_ref[...] * epsf_ref[...]
    base = (yb[:, 0:128] + yb[:, 128:256]
            + yb[:, 256:384] + yb[:, 384:512])
    pre = h0_ref[...] + h1_ref[...] + base + b3_ref[...]
    h = _lrelu(pre)
    out_ref[...] = lax.dot(h, w4t_ref[...], precision=_HI,
                           preferred_element_type=jnp.float32) + b4_ref[...]


def _final(h0, h1, y, epsf, b3r, w4t, b4r):
    N, D = h0.shape
    BN = 1000
    grid = N // BN
    return pl.pallas_call(
        _final_body,
        grid=(grid,),
        in_specs=[
            pl.BlockSpec((BN, D), lambda i: (i, 0)),
            pl.BlockSpec((BN, D), lambda i: (i, 0)),
            pl.BlockSpec((BN, 4 * D), lambda i: (i, 0)),
            pl.BlockSpec((1, 4 * D), lambda i: (0, 0)),
            pl.BlockSpec((1, D), lambda i: (0, 0)),
            pl.BlockSpec((D, D), lambda i: (0, 0)),
            pl.BlockSpec((1, D), lambda i: (0, 0)),
        ],
        out_specs=pl.BlockSpec((BN, D), lambda i: (i, 0)),
        out_shape=jax.ShapeDtypeStruct((N, D), jnp.float32),
    )(h0, h1, y, epsf, b3r, w4t, b4r)


def kernel(x, edge_index, edge_attr, W1, b1, W2, b2, eps, W3, b3, W4, b4):
    N, D = x.shape
    C = W1.shape[0]
    OUT = W3.shape[0]

    w1t = W1.reshape(C * 8, 4).T
    b1s = b1.reshape(1, C * 8)
    w2b = (jnp.eye(C, dtype=jnp.float32)[:, None, :]
           * W2.reshape(C, 8)[:, :, None]).reshape(C * 8, C)
    b2r = b2.reshape(1, C)
    wcat = W3.reshape(OUT, C, D).transpose(2, 1, 0).reshape(D, C * OUT)
    epsf = jnp.repeat(1.0 + eps, OUT).reshape(1, C * OUT)
    b3r = b3.reshape(1, OUT)
    w4t = W4.T
    b4r = b4.reshape(1, OUT)

    wbc = _gates(edge_attr, w1t, b1s, w2b, b2r)
    y = _y_proj(x, wcat)

    E = edge_index.shape[1]
    NW = 32
    EPW = E // NW
    src2 = edge_index[0].reshape(NW, EPW)
    dst2 = edge_index[1].reshape(NW, EPW)
    padi = jnp.zeros((NW, _PAD - EPW), jnp.int32)
    src_p = jnp.concatenate([src2, padi], axis=1).reshape(NW, _PAD // _K, _K)
    dst_p = jnp.concatenate([dst2, padi], axis=1).reshape(NW, _PAD // _K, _K)
    rec = jnp.stack([src_p, dst_p], axis=2).reshape(NW * (_PAD // _K), 2, _K)
    padw = jnp.zeros((NW, _PAD - EPW, 64), jnp.float32)
    wbc_p = jnp.concatenate([wbc.reshape(NW, EPW, 64), padw],
                            axis=1).reshape(NW * _PAD, 64)

    partials = _sc_spmm(y, rec, wbc_p, N, OUT)
    h0 = partials[:N]
    h1 = partials[N:]
    return _final(h0, h1, y, epsf, b3r, w4t, b4r)
